# Initial kernel scaffold; baseline (speedup 1.0000x reference)
import jax
import jax.numpy as jnp
from jax.experimental import pallas as pl


def kernel(feat_id, edge_index, batch_idx, rwse, in_degrees, W_vp, b_vp, W_rwse, b_rwse, deg_emb, eps, W1, b1, W2, b2, gamma, beta):
    raise NotImplementedError("write your pallas kernel here")


# speedup vs baseline: 7.6326x; 7.6326x over previous
import functools

import jax
import jax.numpy as jnp
from jax import lax
from jax.experimental import pallas as pl
from jax.experimental.pallas import tpu as pltpu
from jax.experimental.pallas import tpu_sc as plsc

N = 10000
E = 320000
D = 128
LAYERS = 3
G = 64
FIXED = 128

NC = 2
NS = 16
K = 128
NCHUNK = 80
E_PAD = K * NCHUNK * NC * NS
N_PAD = 10112
ROWS_PT = N_PAD // NS

KG = 80
NCHUNK_G = 4
N_G = KG * NCHUNK_G * NC * NS


def _bdot(a, b):
  return jnp.dot(a.astype(jnp.bfloat16), b.astype(jnp.bfloat16),
                 preferred_element_type=jnp.float32)


def _edge_pass(h, src_t, dst_t, zeros):
  mesh = plsc.VectorSubcoreMesh(core_axis_name="c", subcore_axis_name="s")

  @functools.partial(
      pl.kernel,
      out_type=jax.ShapeDtypeStruct((NC, N_PAD, D), jnp.float32),
      mesh=mesh,
      scratch_types=[
          pltpu.VMEM((NCHUNK, K), jnp.int32),
          pltpu.VMEM((NCHUNK, K), jnp.int32),
          pltpu.VMEM((K, D), jnp.float32),
          pltpu.VMEM_SHARED((N_PAD, D), jnp.float32),
          pltpu.SemaphoreType.DMA,
      ],
  )
  def body(h_hbm, src_hbm, dst_hbm, z_hbm, out_hbm, src_v, dst_v, rows_v,
           agg_sh, sem):
    c = lax.axis_index("c")
    s = lax.axis_index("s")
    pltpu.sync_copy(z_hbm, agg_sh.at[pl.ds(s * ROWS_PT, ROWS_PT)])
    pltpu.sync_copy(src_hbm.at[c, s], src_v)
    pltpu.sync_copy(dst_hbm.at[c, s], dst_v)
    plsc.subcore_barrier()

    def chunk(j, carry):
      pltpu.async_copy(h_hbm.at[src_v.at[j]], rows_v, sem).wait()
      pltpu.sync_copy(rows_v, agg_sh.at[dst_v.at[j]], add=True)
      return carry

    lax.fori_loop(0, NCHUNK, chunk, 0)
    plsc.subcore_barrier()
    pltpu.sync_copy(agg_sh.at[pl.ds(s * ROWS_PT, ROWS_PT)],
                    out_hbm.at[c, pl.ds(s * ROWS_PT, ROWS_PT)])

  return body(h, src_t, dst_t, zeros)


def _row_gather(table, idx_t):
  mesh = plsc.VectorSubcoreMesh(core_axis_name="c", subcore_axis_name="s")

  @functools.partial(
      pl.kernel,
      out_type=jax.ShapeDtypeStruct((NC, NS, NCHUNK_G * KG, D), jnp.float32),
      mesh=mesh,
      scratch_types=[
          pltpu.VMEM((NCHUNK_G, KG), jnp.int32),
          pltpu.VMEM((KG, D), jnp.float32),
          pltpu.SemaphoreType.DMA,
      ],
  )
  def body(tab_hbm, idx_hbm, out_hbm, idx_v, rows_v, sem):
    c = lax.axis_index("c")
    s = lax.axis_index("s")
    pltpu.sync_copy(idx_hbm.at[c, s], idx_v)

    def chunk(j, carry):
      pltpu.async_copy(tab_hbm.at[idx_v.at[j]], rows_v, sem).wait()
      pltpu.sync_copy(rows_v, out_hbm.at[c, s, pl.ds(j * KG, KG)])
      return carry

    lax.fori_loop(0, NCHUNK_G, chunk, 0)

  return body(table, idx_t)


def _encoder(feat2, rwse, W_vp, W_rwse, degrow, bias):
  def body(fid_ref, rwse_ref, wvp_ref, wrwse_ref, degrow_ref, bias_ref,
           out_ref):
    iota = lax.broadcasted_iota(jnp.int32, (N, FIXED), 1)
    hid = lax.rem(fid_ref[...], FIXED)
    oh = jnp.where(hid == iota, 1.0, 0.0)
    acc = _bdot(oh, wvp_ref[...])
    acc = acc + _bdot(rwse_ref[...], wrwse_ref[...])
    out_ref[...] = acc + degrow_ref[...] + bias_ref[...]

  return pl.pallas_call(
      body, out_shape=jax.ShapeDtypeStruct((N, D), jnp.float32),
  )(feat2, rwse, W_vp, W_rwse, degrow, bias)


def _mlp(agg, h, w1, b1, w2, b2, ga, be, ep, bi2, last):
  def body(agg_ref, h_ref, w1_ref, b1_ref, w2_ref, b2_ref, ga_ref, be_ref,
           ep_ref, *rest):
    if last:
      bi_ref, hout_ref, gf_ref = rest
    else:
      (hout_ref,) = rest
    a = agg_ref[0, :N, :] + agg_ref[1, :N, :]
    x = a + (1.0 + ep_ref[0, 0]) * h_ref[...]
    m = jnp.maximum(_bdot(x, w1_ref[...]) + b1_ref[...], 0.0)
    m = _bdot(m, w2_ref[...]) + b2_ref[...]
    mu = jnp.mean(m, axis=0, keepdims=True)
    ctr = m - mu
    var = jnp.mean(ctr * ctr, axis=0, keepdims=True)
    y = ctr / jnp.sqrt(var + 1e-5) * ga_ref[...] + be_ref[...]
    hn = jnp.maximum(y, 0.0)
    hout_ref[...] = hn
    if last:
      giota = lax.broadcasted_iota(jnp.int32, (G, N), 0)
      ogt = jnp.where(bi_ref[...] == giota, 1.0, 0.0)
      sums = jnp.dot(ogt, hn, preferred_element_type=jnp.float32,
                     precision=lax.Precision.HIGHEST)
      counts = jnp.dot(ogt, jnp.ones((N, D), jnp.float32),
                       preferred_element_type=jnp.float32,
                       precision=lax.Precision.HIGHEST)
      gf_ref[...] = sums / jnp.maximum(counts, 1.0)

  if last:
    out_shape = (jax.ShapeDtypeStruct((N, D), jnp.float32),
                 jax.ShapeDtypeStruct((G, D), jnp.float32))
    return pl.pallas_call(body, out_shape=out_shape)(
        agg, h, w1, b1, w2, b2, ga, be, ep, bi2)
  out_shape = jax.ShapeDtypeStruct((N, D), jnp.float32)
  return pl.pallas_call(body, out_shape=out_shape)(
      agg, h, w1, b1, w2, b2, ga, be, ep)


def kernel(feat_id, edge_index, batch_idx, rwse, in_degrees, W_vp, b_vp,
           W_rwse, b_rwse, deg_emb, eps, W1, b1, W2, b2, gamma, beta):
  feat2 = feat_id.astype(jnp.int32).reshape(N, 1)
  bias = (b_vp + b_rwse).reshape(1, D)
  bi2 = batch_idx.astype(jnp.int32).reshape(1, N)

  deg = jnp.clip(in_degrees.astype(jnp.int32), 0, 1000)
  pad_g = (jnp.arange(N_G - N, dtype=jnp.int32) * 37) % 1001
  deg_t = jnp.concatenate([deg, pad_g]).reshape(NC, NS, NCHUNK_G, KG)
  degrow = _row_gather(deg_emb, deg_t).reshape(N_G, D)[:N]

  pad_n = E_PAD - E
  pad_src = (jnp.arange(pad_n, dtype=jnp.int32) * 97) % N
  pad_dst = N + (jnp.arange(pad_n, dtype=jnp.int32) % (N_PAD - N))
  src_t = jnp.concatenate([edge_index[0].astype(jnp.int32), pad_src])
  dst_t = jnp.concatenate([edge_index[1].astype(jnp.int32), pad_dst])
  src_t = src_t.reshape(NC, NS, NCHUNK, K)
  dst_t = dst_t.reshape(NC, NS, NCHUNK, K)
  zeros = jnp.zeros((ROWS_PT, D), jnp.float32)

  h = _encoder(feat2, rwse, W_vp, W_rwse, degrow, bias)
  gf = None
  for l in range(LAYERS):
    agg = _edge_pass(h, src_t, dst_t, zeros)
    last = l == LAYERS - 1
    ep = eps[l].reshape(1, 1)
    res = _mlp(agg, h, W1[l], b1[l].reshape(1, D), W2[l], b2[l].reshape(1, D),
               gamma[l].reshape(1, D), beta[l].reshape(1, D), ep, bi2, last)
    if last:
      h, gf = res
    else:
      h = res
  return (gf, h)


# speedup vs baseline: 10.2151x; 1.3383x over previous
import functools

import jax
import jax.numpy as jnp
from jax import lax
from jax.experimental import pallas as pl
from jax.experimental.pallas import tpu as pltpu
from jax.experimental.pallas import tpu_sc as plsc

N = 10000
E = 320000
D = 128
LAYERS = 3
G = 64
FIXED = 128

NC = 2
NS = 16
K = 128
NCHUNK = 80
E_PAD = K * NCHUNK * NC * NS
N_PAD = 10112
ROWS_PT = N_PAD // NS

KG = 80
NCHUNK_G = 4
N_G = KG * NCHUNK_G * NC * NS


def _bdot(a, b):
  return jnp.dot(a.astype(jnp.bfloat16), b.astype(jnp.bfloat16),
                 preferred_element_type=jnp.float32)


def _edge_pass(h, src_t, dst_t, zeros):
  mesh = plsc.VectorSubcoreMesh(core_axis_name="c", subcore_axis_name="s")

  @functools.partial(
      pl.kernel,
      out_type=jax.ShapeDtypeStruct((NC, N_PAD, D), jnp.float32),
      mesh=mesh,
      scratch_types=[
          pltpu.VMEM((K,), jnp.int32),
          pltpu.VMEM((K,), jnp.int32),
          pltpu.VMEM((K,), jnp.int32),
          pltpu.VMEM((K,), jnp.int32),
          pltpu.VMEM((K, D), jnp.float32),
          pltpu.VMEM((K, D), jnp.float32),
          pltpu.SemaphoreType.DMA,
          pltpu.SemaphoreType.DMA,
          pltpu.SemaphoreType.DMA,
          pltpu.SemaphoreType.DMA,
          pltpu.VMEM_SHARED((N_PAD, D), jnp.float32),
      ],
  )
  def body(h_hbm, src_hbm, dst_hbm, z_hbm, out_hbm, src_a, src_b, dst_a,
           dst_b, rows_a, rows_b, sg_a, sg_b, si_a, si_b, agg_sh):
    c = lax.axis_index("c")
    s = lax.axis_index("s")
    pltpu.sync_copy(z_hbm, agg_sh.at[pl.ds(s * ROWS_PT, ROWS_PT)])
    plsc.subcore_barrier()

    pltpu.sync_copy(src_hbm.at[c, s, 0], src_a)
    pltpu.sync_copy(dst_hbm.at[c, s, 0], dst_a)
    pltpu.async_copy(h_hbm.at[src_a], rows_a, sg_a)
    pltpu.async_copy(src_hbm.at[c, s, 1], src_b, si_b)
    pltpu.async_copy(dst_hbm.at[c, s, 1], dst_b, si_b)

    def step(j, src_c, src_n, dst_c, dst_n, rows_c, rows_n, sg_c, sg_n,
             si_c, si_n):
      @pl.when(j + 1 < NCHUNK)
      def _():
        pltpu.make_async_copy(src_hbm.at[c, s, 0], src_n, si_n).wait()
        pltpu.make_async_copy(dst_hbm.at[c, s, 0], dst_n, si_n).wait()
        pltpu.async_copy(h_hbm.at[src_n], rows_n, sg_n)
      pltpu.make_async_copy(h_hbm.at[pl.ds(0, K)], rows_c, sg_c).wait()
      pltpu.sync_copy(rows_c, agg_sh.at[dst_c], add=True)
      @pl.when(j + 2 < NCHUNK)
      def _():
        pltpu.async_copy(src_hbm.at[c, s, j + 2], src_c, si_c)
        pltpu.async_copy(dst_hbm.at[c, s, j + 2], dst_c, si_c)

    def chunk2(jj, carry):
      j0 = 2 * jj
      step(j0, src_a, src_b, dst_a, dst_b, rows_a, rows_b, sg_a, sg_b,
           si_a, si_b)
      step(j0 + 1, src_b, src_a, dst_b, dst_a, rows_b, rows_a, sg_b, sg_a,
           si_b, si_a)
      return carry

    lax.fori_loop(0, NCHUNK // 2, chunk2, 0)
    plsc.subcore_barrier()
    pltpu.sync_copy(agg_sh.at[pl.ds(s * ROWS_PT, ROWS_PT)],
                    out_hbm.at[c, pl.ds(s * ROWS_PT, ROWS_PT)])

  return body(h, src_t, dst_t, zeros)


def _row_gather(table, idx_t):
  mesh = plsc.VectorSubcoreMesh(core_axis_name="c", subcore_axis_name="s")

  @functools.partial(
      pl.kernel,
      out_type=jax.ShapeDtypeStruct((NC, NS, NCHUNK_G * KG, D), jnp.float32),
      mesh=mesh,
      scratch_types=[
          pltpu.VMEM((NCHUNK_G, KG), jnp.int32),
          pltpu.VMEM((KG, D), jnp.float32),
          pltpu.SemaphoreType.DMA,
      ],
  )
  def body(tab_hbm, idx_hbm, out_hbm, idx_v, rows_v, sem):
    c = lax.axis_index("c")
    s = lax.axis_index("s")
    pltpu.sync_copy(idx_hbm.at[c, s], idx_v)

    def chunk(j, carry):
      pltpu.async_copy(tab_hbm.at[idx_v.at[j]], rows_v, sem).wait()
      pltpu.sync_copy(rows_v, out_hbm.at[c, s, pl.ds(j * KG, KG)])
      return carry

    lax.fori_loop(0, NCHUNK_G, chunk, 0)

  return body(table, idx_t)


def _encoder(feat2, rwse, W_vp, W_rwse, degrow, bias):
  def body(fid_ref, rwse_ref, wvp_ref, wrwse_ref, degrow_ref, bias_ref,
           out_ref):
    iota = lax.broadcasted_iota(jnp.int32, (N, FIXED), 1)
    hid = lax.rem(fid_ref[...], FIXED)
    oh = jnp.where(hid == iota, 1.0, 0.0)
    acc = _bdot(oh, wvp_ref[...])
    acc = acc + _bdot(rwse_ref[...], wrwse_ref[...])
    out_ref[...] = acc + degrow_ref[...] + bias_ref[...]

  return pl.pallas_call(
      body, out_shape=jax.ShapeDtypeStruct((N, D), jnp.float32),
  )(feat2, rwse, W_vp, W_rwse, degrow, bias)


def _mlp(agg, h, w1, b1, w2, b2, ga, be, ep, bi2, last):
  def body(agg_ref, h_ref, w1_ref, b1_ref, w2_ref, b2_ref, ga_ref, be_ref,
           ep_ref, *rest):
    if last:
      bi_ref, hout_ref, gf_ref = rest
    else:
      (hout_ref,) = rest
    a = agg_ref[0, :N, :] + agg_ref[1, :N, :]
    x = a + (1.0 + ep_ref[0, 0]) * h_ref[...]
    m = jnp.maximum(_bdot(x, w1_ref[...]) + b1_ref[...], 0.0)
    m = _bdot(m, w2_ref[...]) + b2_ref[...]
    mu = jnp.mean(m, axis=0, keepdims=True)
    ctr = m - mu
    var = jnp.mean(ctr * ctr, axis=0, keepdims=True)
    y = ctr / jnp.sqrt(var + 1e-5) * ga_ref[...] + be_ref[...]
    hn = jnp.maximum(y, 0.0)
    hout_ref[...] = hn
    if last:
      giota = lax.broadcasted_iota(jnp.int32, (G, N), 0)
      ogt = jnp.where(bi_ref[...] == giota, 1.0, 0.0)
      sums = jnp.dot(ogt, hn, preferred_element_type=jnp.float32,
                     precision=lax.Precision.HIGHEST)
      counts = jnp.dot(ogt, jnp.ones((N, D), jnp.float32),
                       preferred_element_type=jnp.float32,
                       precision=lax.Precision.HIGHEST)
      gf_ref[...] = sums / jnp.maximum(counts, 1.0)

  if last:
    out_shape = (jax.ShapeDtypeStruct((N, D), jnp.float32),
                 jax.ShapeDtypeStruct((G, D), jnp.float32))
    return pl.pallas_call(body, out_shape=out_shape)(
        agg, h, w1, b1, w2, b2, ga, be, ep, bi2)
  out_shape = jax.ShapeDtypeStruct((N, D), jnp.float32)
  return pl.pallas_call(body, out_shape=out_shape)(
      agg, h, w1, b1, w2, b2, ga, be, ep)


def kernel(feat_id, edge_index, batch_idx, rwse, in_degrees, W_vp, b_vp,
           W_rwse, b_rwse, deg_emb, eps, W1, b1, W2, b2, gamma, beta):
  feat2 = feat_id.astype(jnp.int32).reshape(N, 1)
  bias = (b_vp + b_rwse).reshape(1, D)
  bi2 = batch_idx.astype(jnp.int32).reshape(1, N)

  deg = jnp.clip(in_degrees.astype(jnp.int32), 0, 1000)
  pad_g = (jnp.arange(N_G - N, dtype=jnp.int32) * 37) % 1001
  deg_t = jnp.concatenate([deg, pad_g]).reshape(NC, NS, NCHUNK_G, KG)
  degrow = _row_gather(deg_emb, deg_t).reshape(N_G, D)[:N]

  pad_n = E_PAD - E
  pad_src = (jnp.arange(pad_n, dtype=jnp.int32) * 97) % N
  pad_dst = N + (jnp.arange(pad_n, dtype=jnp.int32) % (N_PAD - N))
  src_t = jnp.concatenate([edge_index[0].astype(jnp.int32), pad_src])
  dst_t = jnp.concatenate([edge_index[1].astype(jnp.int32), pad_dst])
  src_t = src_t.reshape(NC, NS, NCHUNK, K)
  dst_t = dst_t.reshape(NC, NS, NCHUNK, K)
  zeros = jnp.zeros((ROWS_PT, D), jnp.float32)

  h = _encoder(feat2, rwse, W_vp, W_rwse, degrow, bias)
  gf = None
  for l in range(LAYERS):
    agg = _edge_pass(h, src_t, dst_t, zeros)
    last = l == LAYERS - 1
    ep = eps[l].reshape(1, 1)
    res = _mlp(agg, h, W1[l], b1[l].reshape(1, D), W2[l], b2[l].reshape(1, D),
               gamma[l].reshape(1, D), beta[l].reshape(1, D), ep, bi2, last)
    if last:
      h, gf = res
    else:
      h = res
  return (gf, h)


# speedup vs baseline: 10.2908x; 1.0074x over previous
import functools

import jax
import jax.numpy as jnp
from jax import lax
from jax.experimental import pallas as pl
from jax.experimental.pallas import tpu as pltpu
from jax.experimental.pallas import tpu_sc as plsc

N = 10000
E = 320000
D = 128
LAYERS = 3
G = 64
FIXED = 128

NC = 2
NS = 16
K = 128
NCHUNK = 80
E_PAD = K * NCHUNK * NC * NS
N_PAD = 10112
ROWS_PT = N_PAD // NS

KG = 80
NCHUNK_G = 4
N_G = KG * NCHUNK_G * NC * NS


def _bdot(a, b):
  return jnp.dot(a.astype(jnp.bfloat16), b.astype(jnp.bfloat16),
                 preferred_element_type=jnp.float32)


def _edge_pass(h, src_t, dst_t, zeros):
  mesh = plsc.VectorSubcoreMesh(core_axis_name="c", subcore_axis_name="s")

  @functools.partial(
      pl.kernel,
      out_type=jax.ShapeDtypeStruct((NC, N_PAD, D), jnp.float32),
      mesh=mesh,
      scratch_types=[
          pltpu.VMEM((K,), jnp.int32),
          pltpu.VMEM((K,), jnp.int32),
          pltpu.VMEM((K,), jnp.int32),
          pltpu.VMEM((K,), jnp.int32),
          pltpu.VMEM((K, D), jnp.float32),
          pltpu.VMEM((K, D), jnp.float32),
          pltpu.SemaphoreType.DMA,
          pltpu.SemaphoreType.DMA,
          pltpu.SemaphoreType.DMA,
          pltpu.SemaphoreType.DMA,
          pltpu.VMEM_SHARED((N_PAD, D), jnp.float32),
      ],
  )
  def body(h_hbm, src_hbm, dst_hbm, z_hbm, out_hbm, src_a, src_b, dst_a,
           dst_b, rows_a, rows_b, sg_a, sg_b, si_a, si_b, agg_sh):
    c = lax.axis_index("c")
    s = lax.axis_index("s")
    pltpu.sync_copy(z_hbm.at[pl.ds(s * ROWS_PT, ROWS_PT)],
                    agg_sh.at[pl.ds(s * ROWS_PT, ROWS_PT)])
    plsc.subcore_barrier()

    pltpu.sync_copy(src_hbm.at[c, s, 0], src_a)
    pltpu.sync_copy(dst_hbm.at[c, s, 0], dst_a)
    pltpu.async_copy(h_hbm.at[src_a], rows_a, sg_a)
    pltpu.async_copy(src_hbm.at[c, s, 1], src_b, si_b)
    pltpu.async_copy(dst_hbm.at[c, s, 1], dst_b, si_b)

    def step(j, src_c, src_n, dst_c, dst_n, rows_c, rows_n, sg_c, sg_n,
             si_c, si_n):
      @pl.when(j + 1 < NCHUNK)
      def _():
        pltpu.make_async_copy(src_hbm.at[c, s, 0], src_n, si_n).wait()
        pltpu.make_async_copy(dst_hbm.at[c, s, 0], dst_n, si_n).wait()
        pltpu.async_copy(h_hbm.at[src_n], rows_n, sg_n)
      pltpu.make_async_copy(h_hbm.at[pl.ds(0, K)], rows_c, sg_c).wait()
      pltpu.sync_copy(rows_c, agg_sh.at[dst_c], add=True)
      @pl.when(j + 2 < NCHUNK)
      def _():
        pltpu.async_copy(src_hbm.at[c, s, j + 2], src_c, si_c)
        pltpu.async_copy(dst_hbm.at[c, s, j + 2], dst_c, si_c)

    def chunk2(jj, carry):
      j0 = 2 * jj
      step(j0, src_a, src_b, dst_a, dst_b, rows_a, rows_b, sg_a, sg_b,
           si_a, si_b)
      step(j0 + 1, src_b, src_a, dst_b, dst_a, rows_b, rows_a, sg_b, sg_a,
           si_b, si_a)
      return carry

    lax.fori_loop(0, NCHUNK // 2, chunk2, 0)
    plsc.subcore_barrier()
    pltpu.sync_copy(agg_sh.at[pl.ds(s * ROWS_PT, ROWS_PT)],
                    out_hbm.at[c, pl.ds(s * ROWS_PT, ROWS_PT)])

  return body(h, src_t, dst_t, zeros)


def _row_gather(table, idx_t):
  mesh = plsc.VectorSubcoreMesh(core_axis_name="c", subcore_axis_name="s")

  @functools.partial(
      pl.kernel,
      out_type=jax.ShapeDtypeStruct((NC, NS, NCHUNK_G * KG, D), jnp.float32),
      mesh=mesh,
      scratch_types=[
          pltpu.VMEM((NCHUNK_G, KG), jnp.int32),
          pltpu.VMEM((KG, D), jnp.float32),
          pltpu.SemaphoreType.DMA,
      ],
  )
  def body(tab_hbm, idx_hbm, out_hbm, idx_v, rows_v, sem):
    c = lax.axis_index("c")
    s = lax.axis_index("s")
    pltpu.sync_copy(idx_hbm.at[c, s], idx_v)

    def chunk(j, carry):
      pltpu.async_copy(tab_hbm.at[idx_v.at[j]], rows_v, sem).wait()
      pltpu.sync_copy(rows_v, out_hbm.at[c, s, pl.ds(j * KG, KG)])
      return carry

    lax.fori_loop(0, NCHUNK_G, chunk, 0)

  return body(table, idx_t)


def _encoder(feat2, rwse, W_vp, W_rwse, degrow, bias):
  def body(fid_ref, rwse_ref, wvp_ref, wrwse_ref, degrow_ref, bias_ref,
           out_ref):
    iota = lax.broadcasted_iota(jnp.int32, (N, FIXED), 1)
    hid = lax.rem(fid_ref[...], FIXED)
    oh = jnp.where(hid == iota, 1.0, 0.0)
    acc = _bdot(oh, wvp_ref[...])
    acc = acc + _bdot(rwse_ref[...], wrwse_ref[...])
    out_ref[...] = acc + degrow_ref[...] + bias_ref[...]

  return pl.pallas_call(
      body, out_shape=jax.ShapeDtypeStruct((N, D), jnp.float32),
  )(feat2, rwse, W_vp, W_rwse, degrow, bias)


def _mlp(agg, h, w1, b1, w2, b2, ga, be, ep, bi2, last):
  def body(agg_ref, h_ref, w1_ref, b1_ref, w2_ref, b2_ref, ga_ref, be_ref,
           ep_ref, *rest):
    if last:
      bi_ref, hout_ref, gf_ref = rest
    else:
      (hout_ref,) = rest
    a = agg_ref[0, :N, :] + agg_ref[1, :N, :]
    x = a + (1.0 + ep_ref[0, 0]) * h_ref[...]
    m = jnp.maximum(_bdot(x, w1_ref[...]) + b1_ref[...], 0.0)
    m = _bdot(m, w2_ref[...]) + b2_ref[...]
    mu = jnp.mean(m, axis=0, keepdims=True)
    ctr = m - mu
    var = jnp.mean(ctr * ctr, axis=0, keepdims=True)
    y = ctr / jnp.sqrt(var + 1e-5) * ga_ref[...] + be_ref[...]
    hn = jnp.maximum(y, 0.0)
    hout_ref[...] = hn
    if last:
      giota = lax.broadcasted_iota(jnp.int32, (G, N), 0)
      ogt = jnp.where(bi_ref[...] == giota, 1.0, 0.0)
      sums = jnp.dot(ogt, hn, preferred_element_type=jnp.float32,
                     precision=lax.Precision.HIGHEST)
      counts = jnp.dot(ogt, jnp.ones((N, D), jnp.float32),
                       preferred_element_type=jnp.float32,
                       precision=lax.Precision.HIGHEST)
      gf_ref[...] = sums / jnp.maximum(counts, 1.0)

  if last:
    out_shape = (jax.ShapeDtypeStruct((N, D), jnp.float32),
                 jax.ShapeDtypeStruct((G, D), jnp.float32))
    return pl.pallas_call(body, out_shape=out_shape)(
        agg, h, w1, b1, w2, b2, ga, be, ep, bi2)
  out_shape = jax.ShapeDtypeStruct((N, D), jnp.float32)
  return pl.pallas_call(body, out_shape=out_shape)(
      agg, h, w1, b1, w2, b2, ga, be, ep)


def kernel(feat_id, edge_index, batch_idx, rwse, in_degrees, W_vp, b_vp,
           W_rwse, b_rwse, deg_emb, eps, W1, b1, W2, b2, gamma, beta):
  feat2 = feat_id.astype(jnp.int32).reshape(N, 1)
  bias = (b_vp + b_rwse).reshape(1, D)
  bi2 = batch_idx.astype(jnp.int32).reshape(1, N)

  deg = jnp.clip(in_degrees.astype(jnp.int32), 0, 1000)
  pad_g = (jnp.arange(N_G - N, dtype=jnp.int32) * 37) % 1001
  deg_t = jnp.concatenate([deg, pad_g]).reshape(NC, NS, NCHUNK_G, KG)
  degrow = _row_gather(deg_emb, deg_t).reshape(N_G, D)[:N]

  pad_n = E_PAD - E
  pad_src = (jnp.arange(pad_n, dtype=jnp.int32) * 97) % N
  pad_dst = N + (jnp.arange(pad_n, dtype=jnp.int32) % (N_PAD - N))
  src_t = jnp.concatenate([edge_index[0].astype(jnp.int32), pad_src])
  dst_t = jnp.concatenate([edge_index[1].astype(jnp.int32), pad_dst])
  src_t = src_t.reshape(NC, NS, NCHUNK, K)
  dst_t = dst_t.reshape(NC, NS, NCHUNK, K)
  zeros = jnp.zeros((N_PAD, D), jnp.float32)

  h = _encoder(feat2, rwse, W_vp, W_rwse, degrow, bias)
  gf = None
  for l in range(LAYERS):
    agg = _edge_pass(h, src_t, dst_t, zeros)
    last = l == LAYERS - 1
    ep = eps[l].reshape(1, 1)
    res = _mlp(agg, h, W1[l], b1[l].reshape(1, D), W2[l], b2[l].reshape(1, D),
               gamma[l].reshape(1, D), beta[l].reshape(1, D), ep, bi2, last)
    if last:
      h, gf = res
    else:
      h = res
  return (gf, h)


# speedup vs baseline: 11.3868x; 1.1065x over previous
import functools

import jax
import jax.numpy as jnp
from jax import lax
from jax.experimental import pallas as pl
from jax.experimental.pallas import tpu as pltpu
from jax.experimental.pallas import tpu_sc as plsc

N = 10000
E = 320000
D = 128
LAYERS = 3
G = 64
FIXED = 128

NC = 2
NS = 16
K = 128
NCHUNK = 80
E_PAD = K * NCHUNK * NC * NS
N_PAD = 10112
ROWS_PT = N_PAD // NS

KG = 80
NCHUNK_G = 4
N_G = KG * NCHUNK_G * NC * NS


def _bdot(a, b):
  return jnp.dot(a.astype(jnp.bfloat16), b.astype(jnp.bfloat16),
                 preferred_element_type=jnp.float32)


def _edge_pass(h, src_t, dst_t, zeros):
  mesh = plsc.VectorSubcoreMesh(core_axis_name="c", subcore_axis_name="s")

  @functools.partial(
      pl.kernel,
      out_type=jax.ShapeDtypeStruct((NC, N_PAD, D), jnp.float32),
      mesh=mesh,
      scratch_types=[
          pltpu.VMEM((K,), jnp.int32),
          pltpu.VMEM((K,), jnp.int32),
          pltpu.VMEM((K,), jnp.int32),
          pltpu.VMEM((K,), jnp.int32),
          pltpu.VMEM((K, D), jnp.float32),
          pltpu.VMEM((K, D), jnp.float32),
          pltpu.SemaphoreType.DMA,
          pltpu.SemaphoreType.DMA,
          pltpu.SemaphoreType.DMA,
          pltpu.SemaphoreType.DMA,
          pltpu.SemaphoreType.DMA,
          pltpu.SemaphoreType.DMA,
          pltpu.VMEM_SHARED((N_PAD, D), jnp.float32),
      ],
  )
  def body(h_hbm, src_hbm, dst_hbm, z_hbm, out_hbm, src_a, src_b, dst_a,
           dst_b, rows_a, rows_b, sg_a, sg_b, si_a, si_b, ss_a, ss_b,
           agg_sh):
    c = lax.axis_index("c")
    s = lax.axis_index("s")
    pltpu.sync_copy(z_hbm.at[pl.ds(s * ROWS_PT, ROWS_PT)],
                    agg_sh.at[pl.ds(s * ROWS_PT, ROWS_PT)])
    plsc.subcore_barrier()

    pltpu.sync_copy(src_hbm.at[c, s, 0], src_a)
    pltpu.sync_copy(dst_hbm.at[c, s, 0], dst_a)
    pltpu.async_copy(h_hbm.at[src_a], rows_a, sg_a)
    pltpu.async_copy(src_hbm.at[c, s, 1], src_b, si_b)
    pltpu.async_copy(dst_hbm.at[c, s, 1], dst_b, si_b)

    def step(j, src_c, src_n, dst_c, dst_n, rows_c, rows_n, sg_c, sg_n,
             si_c, si_n, ss_c, ss_n):
      @pl.when(j + 1 < NCHUNK)
      def _():
        pltpu.make_async_copy(src_hbm.at[c, s, 0], src_n, si_n).wait()
        pltpu.make_async_copy(dst_hbm.at[c, s, 0], dst_n, si_n).wait()

        @pl.when(j >= 1)
        def _():
          pltpu.make_async_copy(h_hbm.at[pl.ds(0, K)], rows_n, ss_n).wait()

        pltpu.async_copy(h_hbm.at[src_n], rows_n, sg_n)
      pltpu.make_async_copy(h_hbm.at[pl.ds(0, K)], rows_c, sg_c).wait()
      pltpu.async_copy(rows_c, agg_sh.at[dst_c], ss_c, add=True)
      @pl.when(j + 2 < NCHUNK)
      def _():
        pltpu.async_copy(src_hbm.at[c, s, j + 2], src_c, si_c)
        pltpu.async_copy(dst_hbm.at[c, s, j + 2], dst_c, si_c)

    def chunk2(jj, carry):
      j0 = 2 * jj
      step(j0, src_a, src_b, dst_a, dst_b, rows_a, rows_b, sg_a, sg_b,
           si_a, si_b, ss_a, ss_b)
      step(j0 + 1, src_b, src_a, dst_b, dst_a, rows_b, rows_a, sg_b, sg_a,
           si_b, si_a, ss_b, ss_a)
      return carry

    lax.fori_loop(0, NCHUNK // 2, chunk2, 0)
    pltpu.make_async_copy(h_hbm.at[pl.ds(0, K)], rows_a, ss_a).wait()
    pltpu.make_async_copy(h_hbm.at[pl.ds(0, K)], rows_b, ss_b).wait()
    plsc.subcore_barrier()
    pltpu.sync_copy(agg_sh.at[pl.ds(s * ROWS_PT, ROWS_PT)],
                    out_hbm.at[c, pl.ds(s * ROWS_PT, ROWS_PT)])

  return body(h, src_t, dst_t, zeros)


def _row_gather(table, idx_t):
  mesh = plsc.VectorSubcoreMesh(core_axis_name="c", subcore_axis_name="s")

  @functools.partial(
      pl.kernel,
      out_type=jax.ShapeDtypeStruct((NC, NS, NCHUNK_G * KG, D), jnp.float32),
      mesh=mesh,
      scratch_types=[
          pltpu.VMEM((NCHUNK_G, KG), jnp.int32),
          pltpu.VMEM((KG, D), jnp.float32),
          pltpu.SemaphoreType.DMA,
      ],
  )
  def body(tab_hbm, idx_hbm, out_hbm, idx_v, rows_v, sem):
    c = lax.axis_index("c")
    s = lax.axis_index("s")
    pltpu.sync_copy(idx_hbm.at[c, s], idx_v)

    def chunk(j, carry):
      pltpu.async_copy(tab_hbm.at[idx_v.at[j]], rows_v, sem).wait()
      pltpu.sync_copy(rows_v, out_hbm.at[c, s, pl.ds(j * KG, KG)])
      return carry

    lax.fori_loop(0, NCHUNK_G, chunk, 0)

  return body(table, idx_t)


def _encoder(feat2, rwse, W_vp, W_rwse, degrow, bias):
  def body(fid_ref, rwse_ref, wvp_ref, wrwse_ref, degrow_ref, bias_ref,
           out_ref):
    iota = lax.broadcasted_iota(jnp.int32, (N, FIXED), 1)
    hid = lax.rem(fid_ref[...], FIXED)
    oh = jnp.where(hid == iota, 1.0, 0.0)
    acc = _bdot(oh, wvp_ref[...])
    acc = acc + _bdot(rwse_ref[...], wrwse_ref[...])
    out_ref[...] = acc + degrow_ref[...] + bias_ref[...]

  return pl.pallas_call(
      body, out_shape=jax.ShapeDtypeStruct((N, D), jnp.float32),
  )(feat2, rwse, W_vp, W_rwse, degrow, bias)


def _mlp(agg, h, w1, b1, w2, b2, ga, be, ep, bi2, last):
  def body(agg_ref, h_ref, w1_ref, b1_ref, w2_ref, b2_ref, ga_ref, be_ref,
           ep_ref, *rest):
    if last:
      bi_ref, hout_ref, gf_ref = rest
    else:
      (hout_ref,) = rest
    a = agg_ref[0, :N, :] + agg_ref[1, :N, :]
    x = a + (1.0 + ep_ref[0, 0]) * h_ref[...]
    m = jnp.maximum(_bdot(x, w1_ref[...]) + b1_ref[...], 0.0)
    m = _bdot(m, w2_ref[...]) + b2_ref[...]
    mu = jnp.mean(m, axis=0, keepdims=True)
    ctr = m - mu
    var = jnp.mean(ctr * ctr, axis=0, keepdims=True)
    y = ctr / jnp.sqrt(var + 1e-5) * ga_ref[...] + be_ref[...]
    hn = jnp.maximum(y, 0.0)
    hout_ref[...] = hn
    if last:
      giota = lax.broadcasted_iota(jnp.int32, (G, N), 0)
      ogt = jnp.where(bi_ref[...] == giota, 1.0, 0.0)
      sums = jnp.dot(ogt, hn, preferred_element_type=jnp.float32,
                     precision=lax.Precision.HIGHEST)
      counts = jnp.dot(ogt, jnp.ones((N, D), jnp.float32),
                       preferred_element_type=jnp.float32,
                       precision=lax.Precision.HIGHEST)
      gf_ref[...] = sums / jnp.maximum(counts, 1.0)

  if last:
    out_shape = (jax.ShapeDtypeStruct((N, D), jnp.float32),
                 jax.ShapeDtypeStruct((G, D), jnp.float32))
    return pl.pallas_call(body, out_shape=out_shape)(
        agg, h, w1, b1, w2, b2, ga, be, ep, bi2)
  out_shape = jax.ShapeDtypeStruct((N, D), jnp.float32)
  return pl.pallas_call(body, out_shape=out_shape)(
      agg, h, w1, b1, w2, b2, ga, be, ep)


def kernel(feat_id, edge_index, batch_idx, rwse, in_degrees, W_vp, b_vp,
           W_rwse, b_rwse, deg_emb, eps, W1, b1, W2, b2, gamma, beta):
  feat2 = feat_id.astype(jnp.int32).reshape(N, 1)
  bias = (b_vp + b_rwse).reshape(1, D)
  bi2 = batch_idx.astype(jnp.int32).reshape(1, N)

  deg = jnp.clip(in_degrees.astype(jnp.int32), 0, 1000)
  pad_g = (jnp.arange(N_G - N, dtype=jnp.int32) * 37) % 1001
  deg_t = jnp.concatenate([deg, pad_g]).reshape(NC, NS, NCHUNK_G, KG)
  degrow = _row_gather(deg_emb, deg_t).reshape(N_G, D)[:N]

  pad_n = E_PAD - E
  pad_src = (jnp.arange(pad_n, dtype=jnp.int32) * 97) % N
  pad_dst = N + (jnp.arange(pad_n, dtype=jnp.int32) % (N_PAD - N))
  src_t = jnp.concatenate([edge_index[0].astype(jnp.int32), pad_src])
  dst_t = jnp.concatenate([edge_index[1].astype(jnp.int32), pad_dst])
  src_t = src_t.reshape(NC, NS, NCHUNK, K)
  dst_t = dst_t.reshape(NC, NS, NCHUNK, K)
  zeros = jnp.zeros((N_PAD, D), jnp.float32)

  h = _encoder(feat2, rwse, W_vp, W_rwse, degrow, bias)
  gf = None
  for l in range(LAYERS):
    agg = _edge_pass(h, src_t, dst_t, zeros)
    last = l == LAYERS - 1
    ep = eps[l].reshape(1, 1)
    res = _mlp(agg, h, W1[l], b1[l].reshape(1, D), W2[l], b2[l].reshape(1, D),
               gamma[l].reshape(1, D), beta[l].reshape(1, D), ep, bi2, last)
    if last:
      h, gf = res
    else:
      h = res
  return (gf, h)


# speedup vs baseline: 11.5018x; 1.0101x over previous
import functools

import jax
import jax.numpy as jnp
from jax import lax
from jax.experimental import pallas as pl
from jax.experimental.pallas import tpu as pltpu
from jax.experimental.pallas import tpu_sc as plsc

N = 10000
E = 320000
D = 128
LAYERS = 3
G = 64
FIXED = 128

NC = 2
NS = 16
K = 128
NCHUNK = 80
E_PAD = K * NCHUNK * NC * NS
N_PAD = 10112
ROWS_PT = N_PAD // NS

KG = 80
NCHUNK_G = 4
N_G = KG * NCHUNK_G * NC * NS


def _bdot(a, b):
  return jnp.dot(a.astype(jnp.bfloat16), b.astype(jnp.bfloat16),
                 preferred_element_type=jnp.float32)


def _edge_pass(h, src_t, dst_t, zeros):
  mesh = plsc.VectorSubcoreMesh(core_axis_name="c", subcore_axis_name="s")

  @functools.partial(
      pl.kernel,
      out_type=jax.ShapeDtypeStruct((NC, N_PAD, D), jnp.float32),
      mesh=mesh,
      scratch_types=[
          pltpu.VMEM((K,), jnp.int32),
          pltpu.VMEM((K,), jnp.int32),
          pltpu.VMEM((K,), jnp.int32),
          pltpu.VMEM((K,), jnp.int32),
          pltpu.VMEM((K, D), jnp.float32),
          pltpu.VMEM((K, D), jnp.float32),
          pltpu.SemaphoreType.DMA,
          pltpu.SemaphoreType.DMA,
          pltpu.SemaphoreType.DMA,
          pltpu.SemaphoreType.DMA,
          pltpu.SemaphoreType.DMA,
          pltpu.SemaphoreType.DMA,
          pltpu.VMEM_SHARED((N_PAD, D), jnp.float32),
      ],
  )
  def body(h_hbm, src_hbm, dst_hbm, z_hbm, out_hbm, src_a, src_b, dst_a,
           dst_b, rows_a, rows_b, sg_a, sg_b, si_a, si_b, ss_a, ss_b,
           agg_sh):
    c = lax.axis_index("c")
    s = lax.axis_index("s")
    pltpu.sync_copy(z_hbm.at[pl.ds(s * ROWS_PT, ROWS_PT)],
                    agg_sh.at[pl.ds(s * ROWS_PT, ROWS_PT)])
    plsc.subcore_barrier()

    pltpu.sync_copy(src_hbm.at[c, s, 0], src_a)
    pltpu.sync_copy(dst_hbm.at[c, s, 0], dst_a)
    pltpu.async_copy(h_hbm.at[src_a], rows_a, sg_a)
    pltpu.async_copy(src_hbm.at[c, s, 1], src_b, si_b)
    pltpu.async_copy(dst_hbm.at[c, s, 1], dst_b, si_b)

    def step(j, src_c, src_n, dst_c, dst_n, rows_c, rows_n, sg_c, sg_n,
             si_c, si_n, ss_c, ss_n):
      @pl.when(j + 1 < NCHUNK)
      def _():
        pltpu.make_async_copy(src_hbm.at[c, s, 0], src_n, si_n).wait()
        pltpu.make_async_copy(dst_hbm.at[c, s, 0], dst_n, si_n).wait()

        @pl.when(j >= 1)
        def _():
          pltpu.make_async_copy(h_hbm.at[pl.ds(0, K)], rows_n, ss_n).wait()

        pltpu.async_copy(h_hbm.at[src_n], rows_n, sg_n)
      pltpu.make_async_copy(h_hbm.at[pl.ds(0, K)], rows_c, sg_c).wait()
      pltpu.async_copy(rows_c, agg_sh.at[dst_c], ss_c, add=True)
      @pl.when(j + 2 < NCHUNK)
      def _():
        pltpu.async_copy(src_hbm.at[c, s, j + 2], src_c, si_c)
        pltpu.async_copy(dst_hbm.at[c, s, j + 2], dst_c, si_c)

    def chunk2(jj, carry):
      j0 = 2 * jj
      step(j0, src_a, src_b, dst_a, dst_b, rows_a, rows_b, sg_a, sg_b,
           si_a, si_b, ss_a, ss_b)
      step(j0 + 1, src_b, src_a, dst_b, dst_a, rows_b, rows_a, sg_b, sg_a,
           si_b, si_a, ss_b, ss_a)
      return carry

    lax.fori_loop(0, NCHUNK // 2, chunk2, 0)
    pltpu.make_async_copy(h_hbm.at[pl.ds(0, K)], rows_a, ss_a).wait()
    pltpu.make_async_copy(h_hbm.at[pl.ds(0, K)], rows_b, ss_b).wait()
    plsc.subcore_barrier()
    pltpu.sync_copy(agg_sh.at[pl.ds(s * ROWS_PT, ROWS_PT)],
                    out_hbm.at[c, pl.ds(s * ROWS_PT, ROWS_PT)])

  return body(h, src_t, dst_t, zeros)


def _row_gather(table, idx_t):
  mesh = plsc.VectorSubcoreMesh(core_axis_name="c", subcore_axis_name="s")

  @functools.partial(
      pl.kernel,
      out_type=jax.ShapeDtypeStruct((NC, NS, NCHUNK_G * KG, D), jnp.float32),
      mesh=mesh,
      scratch_types=[
          pltpu.VMEM((NCHUNK_G, KG), jnp.int32),
          pltpu.VMEM((KG, D), jnp.float32),
          pltpu.SemaphoreType.DMA,
      ],
  )
  def body(tab_hbm, idx_hbm, out_hbm, idx_v, rows_v, sem):
    c = lax.axis_index("c")
    s = lax.axis_index("s")
    pltpu.sync_copy(idx_hbm.at[c, s], idx_v)

    def chunk(j, carry):
      pltpu.async_copy(tab_hbm.at[idx_v.at[j]], rows_v, sem).wait()
      pltpu.sync_copy(rows_v, out_hbm.at[c, s, pl.ds(j * KG, KG)])
      return carry

    lax.fori_loop(0, NCHUNK_G, chunk, 0)

  return body(table, idx_t)


def _encoder(feat2, rwse, W_vp, W_rwse, degrow_full, bias):
  def body(fid_ref, rwse_ref, wvp_ref, wrwse_ref, degrow_ref, bias_ref,
           out_ref):
    iota = lax.broadcasted_iota(jnp.int32, (N, FIXED), 1)
    hid = lax.rem(fid_ref[...], FIXED)
    oh = jnp.where(hid == iota, 1.0, 0.0)
    acc = _bdot(oh, wvp_ref[...])
    acc = acc + _bdot(rwse_ref[...], wrwse_ref[...])
    out_ref[...] = acc + degrow_ref[:N, :] + bias_ref[...]

  return pl.pallas_call(
      body, out_shape=jax.ShapeDtypeStruct((N, D), jnp.float32),
  )(feat2, rwse, W_vp, W_rwse, degrow_full, bias)


def _mlp(agg, h, w1, b1, w2, b2, ga, be, ep, bi2, last):
  def body(agg_ref, h_ref, w1_ref, b1_ref, w2_ref, b2_ref, ga_ref, be_ref,
           ep_ref, *rest):
    if last:
      bi_ref, hout_ref, gf_ref = rest
    else:
      (hout_ref,) = rest
    a = agg_ref[0, :N, :] + agg_ref[1, :N, :]
    x = a + (1.0 + ep_ref[0, 0]) * h_ref[...]
    m = jnp.maximum(_bdot(x, w1_ref[...]) + b1_ref[...], 0.0)
    m = _bdot(m, w2_ref[...]) + b2_ref[...]
    mu = jnp.mean(m, axis=0, keepdims=True)
    ctr = m - mu
    var = jnp.mean(ctr * ctr, axis=0, keepdims=True)
    y = ctr / jnp.sqrt(var + 1e-5) * ga_ref[...] + be_ref[...]
    hn = jnp.maximum(y, 0.0)
    hout_ref[...] = hn
    if last:
      giota = lax.broadcasted_iota(jnp.int32, (G, N), 0)
      ogt = jnp.where(bi_ref[...] == giota, 1.0, 0.0)
      sums = jnp.dot(ogt, hn, preferred_element_type=jnp.float32,
                     precision=lax.Precision.HIGHEST)
      counts = jnp.dot(ogt, jnp.ones((N, D), jnp.float32),
                       preferred_element_type=jnp.float32,
                       precision=lax.Precision.HIGHEST)
      gf_ref[...] = sums / jnp.maximum(counts, 1.0)

  if last:
    out_shape = (jax.ShapeDtypeStruct((N, D), jnp.float32),
                 jax.ShapeDtypeStruct((G, D), jnp.float32))
    return pl.pallas_call(body, out_shape=out_shape)(
        agg, h, w1, b1, w2, b2, ga, be, ep, bi2)
  out_shape = jax.ShapeDtypeStruct((N, D), jnp.float32)
  return pl.pallas_call(body, out_shape=out_shape)(
      agg, h, w1, b1, w2, b2, ga, be, ep)


def kernel(feat_id, edge_index, batch_idx, rwse, in_degrees, W_vp, b_vp,
           W_rwse, b_rwse, deg_emb, eps, W1, b1, W2, b2, gamma, beta):
  feat2 = feat_id.astype(jnp.int32).reshape(N, 1)
  bias = (b_vp + b_rwse).reshape(1, D)
  bi2 = batch_idx.astype(jnp.int32).reshape(1, N)

  deg = jnp.clip(in_degrees.astype(jnp.int32), 0, 1000)
  pad_g = (jnp.arange(N_G - N, dtype=jnp.int32) * 37) % 1001
  deg_t = jnp.concatenate([deg, pad_g]).reshape(NC, NS, NCHUNK_G, KG)
  degrow = _row_gather(deg_emb, deg_t).reshape(N_G, D)

  pad_n = E_PAD - E
  pad_src = (jnp.arange(pad_n, dtype=jnp.int32) * 97) % N
  pad_dst = N + (jnp.arange(pad_n, dtype=jnp.int32) % (N_PAD - N))
  src_t = jnp.concatenate([edge_index[0].astype(jnp.int32), pad_src])
  dst_t = jnp.concatenate([edge_index[1].astype(jnp.int32), pad_dst])
  src_t = src_t.reshape(NC, NS, NCHUNK, K)
  dst_t = dst_t.reshape(NC, NS, NCHUNK, K)
  zeros = jnp.zeros((N_PAD, D), jnp.float32)

  h = _encoder(feat2, rwse, W_vp, W_rwse, degrow, bias)
  gf = None
  for l in range(LAYERS):
    agg = _edge_pass(h, src_t, dst_t, zeros)
    last = l == LAYERS - 1
    ep = eps[l].reshape(1, 1)
    res = _mlp(agg, h, W1[l], b1[l].reshape(1, D), W2[l], b2[l].reshape(1, D),
               gamma[l].reshape(1, D), beta[l].reshape(1, D), ep, bi2, last)
    if last:
      h, gf = res
    else:
      h = res
  return (gf, h)
